# Initial kernel scaffold; baseline (speedup 1.0000x reference)
import functools

import jax
import jax.numpy as jnp
from jax import lax
from jax.experimental import pallas as pl
from jax.experimental.pallas import tpu as pltpu
from jax.experimental.pallas import tpu_sc as plsc

N = 10000
NP = 10240
D = 128
E = 320000
NC = 2
NS = 16
NW = NC * NS
EPW = E // NW
K = 80
NCHUNK = EPW // K
RPT = NP // NS

_MESH = plsc.VectorSubcoreMesh(core_axis_name="c", subcore_axis_name="s")


@functools.partial(
    pl.kernel,
    mesh=_MESH,
    out_type=jax.ShapeDtypeStruct((NC, NP), jnp.float32),
    scratch_types=[
        pltpu.VMEM((K,), jnp.int32),
        pltpu.VMEM((K,), jnp.float32),
        pltpu.VMEM_SHARED((NP,), jnp.float32),
    ],
)
def _deg_sc(dst_hbm, zeros_hbm, out_hbm, didx, ones_v, acc):
    c = lax.axis_index("c")
    s = lax.axis_index("s")
    wid = s * NC + c

    def fill_ones(i, carry):
        ones_v[pl.ds(i * 16, 16)] = jnp.ones((16,), jnp.float32)
        return carry

    lax.fori_loop(0, K // 16, fill_ones, 0)
    pltpu.sync_copy(zeros_hbm, acc.at[pl.ds(s * RPT, RPT)])
    plsc.subcore_barrier()

    def body(i, carry):
        base = wid * EPW + i * K
        pltpu.sync_copy(dst_hbm.at[pl.ds(base, K)], didx)
        pltpu.sync_copy(ones_v, acc.at[didx], add=True)
        return carry

    lax.fori_loop(0, NCHUNK, body, 0)
    plsc.subcore_barrier()
    pltpu.sync_copy(acc.at[pl.ds(s * RPT, RPT)], out_hbm.at[c, pl.ds(s * RPT, RPT)])


@functools.partial(
    pl.kernel,
    mesh=_MESH,
    out_type=jax.ShapeDtypeStruct((NC, NP, D), jnp.float32),
    scratch_types=[
        pltpu.VMEM((K,), jnp.int32),
        pltpu.VMEM((K,), jnp.int32),
        pltpu.VMEM((K, D), jnp.float32),
        pltpu.VMEM_SHARED((NP, D), jnp.float32),
        pltpu.SemaphoreType.DMA,
    ],
)
def _agg_sc(y_hbm, src_hbm, dst_hbm, zeros_hbm, out_hbm, sidx, didx, rows, acc, sem):
    c = lax.axis_index("c")
    s = lax.axis_index("s")
    wid = s * NC + c

    pltpu.sync_copy(zeros_hbm, acc.at[pl.ds(s * RPT, RPT)])
    plsc.subcore_barrier()

    def body(i, carry):
        base = wid * EPW + i * K
        pltpu.sync_copy(src_hbm.at[pl.ds(base, K)], sidx)
        pltpu.async_copy(y_hbm.at[sidx], rows, sem).wait()
        pltpu.sync_copy(dst_hbm.at[pl.ds(base, K)], didx)
        pltpu.sync_copy(rows, acc.at[didx], add=True)
        return carry

    lax.fori_loop(0, NCHUNK, body, 0)
    plsc.subcore_barrier()
    pltpu.sync_copy(
        acc.at[pl.ds(s * RPT, RPT)], out_hbm.at[c, pl.ds(s * RPT, RPT)]
    )


def _prep_body(degs_ref, x_ref, dinv_ref, y1_ref):
    dinv = lax.rsqrt(degs_ref[...])
    dinv_ref[...] = dinv
    y1_ref[...] = x_ref[...] * dinv[:N]


def _prep_tc(degs, x):
    return pl.pallas_call(
        _prep_body,
        out_shape=(
            jax.ShapeDtypeStruct((NP, 1), jnp.float32),
            jax.ShapeDtypeStruct((N, D), jnp.float32),
        ),
    )(degs, x)


_BLK = 500
_NBLK = N // _BLK


def _mid_body(y1_ref, pp_ref, dinv_ref, w1_ref, b1_ref, pw1_ref, w2_ref, y2_ref):
    z = y1_ref[...] + pp_ref[0] + pp_ref[1]
    zh = z * dinv_ref[...]
    h = jnp.dot(zh, w1_ref[...], preferred_element_type=jnp.float32) + b1_ref[...]
    h = jnp.where(h >= 0, h, h * pw1_ref[...])
    g = jnp.dot(h, w2_ref[...], preferred_element_type=jnp.float32)
    y2_ref[...] = g * dinv_ref[...]


def _mid_tc(y1, pp, dinv, w1, b1, pw1, w2):
    dh = w1.shape[1]
    return pl.pallas_call(
        _mid_body,
        grid=(_NBLK,),
        in_specs=[
            pl.BlockSpec((_BLK, D), lambda i: (i, 0)),
            pl.BlockSpec((NC, _BLK, D), lambda i: (0, i, 0)),
            pl.BlockSpec((_BLK, 1), lambda i: (i, 0)),
            pl.BlockSpec((D, dh), lambda i: (0, 0)),
            pl.BlockSpec((1, dh), lambda i: (0, 0)),
            pl.BlockSpec((1, dh), lambda i: (0, 0)),
            pl.BlockSpec((dh, D), lambda i: (0, 0)),
        ],
        out_specs=pl.BlockSpec((_BLK, D), lambda i: (i, 0)),
        out_shape=jax.ShapeDtypeStruct((N, D), jnp.float32),
    )(y1, pp, dinv, w1, b1, pw1, w2)


def _fin_body(y2_ref, qq_ref, dinv_ref, b2_ref, pw2_ref, o_ref):
    z = y2_ref[...] + qq_ref[0] + qq_ref[1]
    h = z * dinv_ref[...] + b2_ref[...]
    o_ref[...] = jnp.where(h >= 0, h, h * pw2_ref[...])


def _fin_tc(y2, qq, dinv, b2, pw2):
    return pl.pallas_call(
        _fin_body,
        grid=(_NBLK,),
        in_specs=[
            pl.BlockSpec((_BLK, D), lambda i: (i, 0)),
            pl.BlockSpec((NC, _BLK, D), lambda i: (0, i, 0)),
            pl.BlockSpec((_BLK, 1), lambda i: (i, 0)),
            pl.BlockSpec((1, D), lambda i: (0, 0)),
            pl.BlockSpec((1, D), lambda i: (0, 0)),
        ],
        out_specs=pl.BlockSpec((_BLK, D), lambda i: (i, 0)),
        out_shape=jax.ShapeDtypeStruct((N, D), jnp.float32),
    )(y2, qq, dinv, b2, pw2)


@jax.jit
def kernel(x, edge_index, W1, b1, p1, W2, b2, p2):
    src = edge_index[0].astype(jnp.int32)
    dst = edge_index[1].astype(jnp.int32)
    zeros1 = jnp.zeros((RPT,), jnp.float32)
    zeros2 = jnp.zeros((RPT, D), jnp.float32)

    degp = _deg_sc(dst, zeros1)
    degs = (degp[0] + degp[1] + 1.0)[:, None]
    dinv, y1 = _prep_tc(degs, x)

    pp = _agg_sc(y1, src, dst, zeros2)
    y2 = _mid_tc(y1, pp, dinv, W1, b1[None, :], p1[None, :], W2)

    qq = _agg_sc(y2, src, dst, zeros2)
    return _fin_tc(y2, qq, dinv, b2[None, :], p2[None, :])


# speedup vs baseline: 17.5470x; 17.5470x over previous
import functools

import jax
import jax.numpy as jnp
from jax import lax
from jax.experimental import pallas as pl
from jax.experimental.pallas import tpu as pltpu
from jax.experimental.pallas import tpu_sc as plsc

N = 10000
NP = 10240
D = 128
E = 320000
NC = 2
NS = 16
NW = NC * NS
EPW = E // NW
K = 80
NCHUNK = EPW // K
RPT = NP // NS

def _deg_body(dst_hbm, zeros_hbm, out_hbm, didx, ones_v, acc):
    c = lax.axis_index("c")
    s = lax.axis_index("s")
    wid = s * NC + c

    def fill_ones(i, carry):
        ones_v[pl.ds(i * 16, 16)] = jnp.ones((16,), jnp.float32)
        return carry

    lax.fori_loop(0, K // 16, fill_ones, 0)
    pltpu.sync_copy(zeros_hbm, acc.at[pl.ds(s * RPT, RPT)])
    plsc.subcore_barrier()

    def body(i, carry):
        base = wid * EPW + i * K
        pltpu.sync_copy(dst_hbm.at[pl.ds(base, K)], didx)
        pltpu.sync_copy(ones_v, acc.at[didx], add=True)
        return carry

    lax.fori_loop(0, NCHUNK, body, 0)
    plsc.subcore_barrier()
    pltpu.sync_copy(acc.at[pl.ds(s * RPT, RPT)], out_hbm.at[c, pl.ds(s * RPT, RPT)])


@functools.cache
def _deg_sc():
    return pl.kernel(
        _deg_body,
        mesh=plsc.VectorSubcoreMesh(core_axis_name="c", subcore_axis_name="s"),
        out_type=jax.ShapeDtypeStruct((NC, NP), jnp.float32),
        scratch_types=[
            pltpu.VMEM((K,), jnp.int32),
            pltpu.VMEM((K,), jnp.float32),
            pltpu.VMEM_SHARED((NP,), jnp.float32),
        ],
    )


def _agg_body(y_hbm, src_hbm, dst_hbm, zeros_hbm, out_hbm, sidx, didx, rows, acc, sem):
    c = lax.axis_index("c")
    s = lax.axis_index("s")
    wid = s * NC + c

    pltpu.sync_copy(zeros_hbm, acc.at[pl.ds(s * RPT, RPT)])
    plsc.subcore_barrier()

    def body(i, carry):
        base = wid * EPW + i * K
        pltpu.sync_copy(src_hbm.at[pl.ds(base, K)], sidx)
        pltpu.async_copy(y_hbm.at[sidx], rows, sem).wait()
        pltpu.sync_copy(dst_hbm.at[pl.ds(base, K)], didx)
        pltpu.sync_copy(rows, acc.at[didx], add=True)
        return carry

    lax.fori_loop(0, NCHUNK, body, 0)
    plsc.subcore_barrier()
    pltpu.sync_copy(
        acc.at[pl.ds(s * RPT, RPT)], out_hbm.at[c, pl.ds(s * RPT, RPT)]
    )


@functools.cache
def _agg_sc():
    return pl.kernel(
        _agg_body,
        mesh=plsc.VectorSubcoreMesh(core_axis_name="c", subcore_axis_name="s"),
        out_type=jax.ShapeDtypeStruct((NC, NP, D), jnp.float32),
        scratch_types=[
            pltpu.VMEM((K,), jnp.int32),
            pltpu.VMEM((K,), jnp.int32),
            pltpu.VMEM((K, D), jnp.float32),
            pltpu.VMEM_SHARED((NP, D), jnp.float32),
            pltpu.SemaphoreType.DMA,
        ],
    )


def _prep_body(degs_ref, x_ref, dinv_ref, y1_ref):
    dinv = lax.rsqrt(degs_ref[...])
    dinv_ref[...] = dinv
    y1_ref[...] = x_ref[...] * dinv[:N]


def _prep_tc(degs, x):
    return pl.pallas_call(
        _prep_body,
        out_shape=(
            jax.ShapeDtypeStruct((NP, 1), jnp.float32),
            jax.ShapeDtypeStruct((N, D), jnp.float32),
        ),
    )(degs, x)


_BLK = 1000
_NBLK = N // _BLK


def _mid_body(y1_ref, pp_ref, dinv_ref, w1_ref, b1_ref, pw1_ref, w2_ref, y2_ref):
    z = y1_ref[...] + pp_ref[0] + pp_ref[1]
    zh = z * dinv_ref[...]
    h = jnp.dot(zh, w1_ref[...], preferred_element_type=jnp.float32) + b1_ref[...]
    h = jnp.where(h >= 0, h, h * pw1_ref[...])
    g = jnp.dot(h, w2_ref[...], preferred_element_type=jnp.float32)
    y2_ref[...] = g * dinv_ref[...]


def _mid_tc(y1, pp, dinv, w1, b1, pw1, w2):
    dh = w1.shape[1]
    return pl.pallas_call(
        _mid_body,
        grid=(_NBLK,),
        in_specs=[
            pl.BlockSpec((_BLK, D), lambda i: (i, 0)),
            pl.BlockSpec((NC, _BLK, D), lambda i: (0, i, 0)),
            pl.BlockSpec((_BLK, 1), lambda i: (i, 0)),
            pl.BlockSpec((D, dh), lambda i: (0, 0)),
            pl.BlockSpec((1, dh), lambda i: (0, 0)),
            pl.BlockSpec((1, dh), lambda i: (0, 0)),
            pl.BlockSpec((dh, D), lambda i: (0, 0)),
        ],
        out_specs=pl.BlockSpec((_BLK, D), lambda i: (i, 0)),
        out_shape=jax.ShapeDtypeStruct((N, D), jnp.float32),
    )(y1, pp, dinv, w1, b1, pw1, w2)


def _fin_body(y2_ref, qq_ref, dinv_ref, b2_ref, pw2_ref, o_ref):
    z = y2_ref[...] + qq_ref[0] + qq_ref[1]
    h = z * dinv_ref[...] + b2_ref[...]
    o_ref[...] = jnp.where(h >= 0, h, h * pw2_ref[...])


def _fin_tc(y2, qq, dinv, b2, pw2):
    return pl.pallas_call(
        _fin_body,
        grid=(_NBLK,),
        in_specs=[
            pl.BlockSpec((_BLK, D), lambda i: (i, 0)),
            pl.BlockSpec((NC, _BLK, D), lambda i: (0, i, 0)),
            pl.BlockSpec((_BLK, 1), lambda i: (i, 0)),
            pl.BlockSpec((1, D), lambda i: (0, 0)),
            pl.BlockSpec((1, D), lambda i: (0, 0)),
        ],
        out_specs=pl.BlockSpec((_BLK, D), lambda i: (i, 0)),
        out_shape=jax.ShapeDtypeStruct((N, D), jnp.float32),
    )(y2, qq, dinv, b2, pw2)


@jax.jit
def kernel(x, edge_index, W1, b1, p1, W2, b2, p2):
    src = edge_index[0].astype(jnp.int32)
    dst = edge_index[1].astype(jnp.int32)
    zeros1 = jnp.zeros((RPT,), jnp.float32)
    zeros2 = jnp.zeros((RPT, D), jnp.float32)

    degp = _deg_sc()(dst, zeros1)
    degs = (degp[0] + degp[1] + 1.0)[:, None]
    dinv, y1 = _prep_tc(degs, x)

    pp = _agg_sc()(y1, src, dst, zeros2)
    y2 = _mid_tc(y1, pp, dinv, W1, b1[None, :], p1[None, :], W2)

    qq = _agg_sc()(y2, src, dst, zeros2)
    return _fin_tc(y2, qq, dinv, b2[None, :], p2[None, :])


# speedup vs baseline: 35.9504x; 2.0488x over previous
import functools

import jax
import jax.numpy as jnp
from jax import lax
from jax.experimental import pallas as pl
from jax.experimental.pallas import tpu as pltpu
from jax.experimental.pallas import tpu_sc as plsc

N = 10000
NP = 10240
D = 128
E = 320000
NC = 2
NS = 16
NW = NC * NS
EPW = E // NW
K = 80
NCHUNK = EPW // K
RPT = NP // NS

def _deg_body(dst_hbm, zeros_hbm, out_hbm, didx, ones_v, acc, dsem):
    c = lax.axis_index("c")
    s = lax.axis_index("s")
    wid = s * NC + c

    def fill_ones(i, carry):
        ones_v[pl.ds(i * 16, 16)] = jnp.ones((16,), jnp.float32)
        return carry

    lax.fori_loop(0, K // 16, fill_ones, 0)
    pltpu.sync_copy(zeros_hbm, acc.at[pl.ds(s * RPT, RPT)])
    pltpu.async_copy(dst_hbm.at[wid, 0], didx.at[0], dsem.at[0])
    plsc.subcore_barrier()

    def body(i, carry):
        b = lax.rem(i, 2)
        pltpu.make_async_copy(dst_hbm.at[wid, i], didx.at[b], dsem.at[b]).wait()

        @pl.when(i + 1 < NCHUNK)
        def _():
            pltpu.async_copy(dst_hbm.at[wid, i + 1], didx.at[1 - b], dsem.at[1 - b])

        pltpu.sync_copy(ones_v, acc.at[didx.at[b]], add=True)
        return carry

    lax.fori_loop(0, NCHUNK, body, 0)
    plsc.subcore_barrier()
    pltpu.sync_copy(acc.at[pl.ds(s * RPT, RPT)], out_hbm.at[c, pl.ds(s * RPT, RPT)])


@functools.cache
def _deg_sc():
    return pl.kernel(
        _deg_body,
        mesh=plsc.VectorSubcoreMesh(core_axis_name="c", subcore_axis_name="s"),
        out_type=jax.ShapeDtypeStruct((NC, NP), jnp.float32),
        scratch_types=[
            pltpu.VMEM((2, K), jnp.int32),
            pltpu.VMEM((K,), jnp.float32),
            pltpu.VMEM_SHARED((NP,), jnp.float32),
            pltpu.SemaphoreType.DMA((2,)),
        ],
    )


NBUF = 2


def _agg_body(
    y_hbm, src_hbm, dst_hbm, zeros_hbm, out_hbm, sidx, didx, rows, acc, gsem, dsem
):
    c = lax.axis_index("c")
    s = lax.axis_index("s")
    wid = s * NC + c

    pltpu.sync_copy(src_hbm.at[wid], sidx)
    pltpu.sync_copy(zeros_hbm, acc.at[pl.ds(s * RPT, RPT)])
    for j in range(NBUF):
        pltpu.async_copy(dst_hbm.at[wid, j], didx.at[j], dsem.at[j])
        pltpu.async_copy(y_hbm.at[sidx.at[j]], rows.at[j], gsem.at[j])
    plsc.subcore_barrier()

    def body(i, carry):
        b = lax.rem(i, NBUF)
        pltpu.make_async_copy(y_hbm.at[sidx.at[i]], rows.at[b], gsem.at[b]).wait()
        pltpu.make_async_copy(dst_hbm.at[wid, i], didx.at[b], dsem.at[b]).wait()
        pltpu.sync_copy(rows.at[b], acc.at[didx.at[b]], add=True)

        @pl.when(i + NBUF < NCHUNK)
        def _():
            pltpu.async_copy(dst_hbm.at[wid, i + NBUF], didx.at[b], dsem.at[b])
            pltpu.async_copy(y_hbm.at[sidx.at[i + NBUF]], rows.at[b], gsem.at[b])

        return carry

    lax.fori_loop(0, NCHUNK, body, 0)
    plsc.subcore_barrier()
    pltpu.sync_copy(
        acc.at[pl.ds(s * RPT, RPT)], out_hbm.at[c, pl.ds(s * RPT, RPT)]
    )


@functools.cache
def _agg_sc():
    return pl.kernel(
        _agg_body,
        mesh=plsc.VectorSubcoreMesh(core_axis_name="c", subcore_axis_name="s"),
        out_type=jax.ShapeDtypeStruct((NC, NP, D), jnp.float32),
        scratch_types=[
            pltpu.VMEM((NCHUNK, K), jnp.int32),
            pltpu.VMEM((NBUF, K), jnp.int32),
            pltpu.VMEM((NBUF, K, D), jnp.float32),
            pltpu.VMEM_SHARED((NP, D), jnp.float32),
            pltpu.SemaphoreType.DMA((NBUF,)),
            pltpu.SemaphoreType.DMA((NBUF,)),
        ],
    )


def _prep_body(degs_ref, x_ref, dinv_ref, y1_ref):
    dinv = lax.rsqrt(degs_ref[...])
    dinv_ref[...] = dinv
    y1_ref[...] = x_ref[...] * dinv[:N]


def _prep_tc(degs, x):
    return pl.pallas_call(
        _prep_body,
        out_shape=(
            jax.ShapeDtypeStruct((NP, 1), jnp.float32),
            jax.ShapeDtypeStruct((N, D), jnp.float32),
        ),
    )(degs, x)


_BLK = 1000
_NBLK = N // _BLK


def _mid_body(y1_ref, pp_ref, dinv_ref, w1_ref, b1_ref, pw1_ref, w2_ref, y2_ref):
    z = y1_ref[...] + pp_ref[0] + pp_ref[1]
    zh = z * dinv_ref[...]
    h = jnp.dot(zh, w1_ref[...], preferred_element_type=jnp.float32) + b1_ref[...]
    h = jnp.where(h >= 0, h, h * pw1_ref[...])
    g = jnp.dot(h, w2_ref[...], preferred_element_type=jnp.float32)
    y2_ref[...] = g * dinv_ref[...]


def _mid_tc(y1, pp, dinv, w1, b1, pw1, w2):
    dh = w1.shape[1]
    return pl.pallas_call(
        _mid_body,
        grid=(_NBLK,),
        in_specs=[
            pl.BlockSpec((_BLK, D), lambda i: (i, 0)),
            pl.BlockSpec((NC, _BLK, D), lambda i: (0, i, 0)),
            pl.BlockSpec((_BLK, 1), lambda i: (i, 0)),
            pl.BlockSpec((D, dh), lambda i: (0, 0)),
            pl.BlockSpec((1, dh), lambda i: (0, 0)),
            pl.BlockSpec((1, dh), lambda i: (0, 0)),
            pl.BlockSpec((dh, D), lambda i: (0, 0)),
        ],
        out_specs=pl.BlockSpec((_BLK, D), lambda i: (i, 0)),
        out_shape=jax.ShapeDtypeStruct((N, D), jnp.float32),
    )(y1, pp, dinv, w1, b1, pw1, w2)


def _fin_body(y2_ref, qq_ref, dinv_ref, b2_ref, pw2_ref, o_ref):
    z = y2_ref[...] + qq_ref[0] + qq_ref[1]
    h = z * dinv_ref[...] + b2_ref[...]
    o_ref[...] = jnp.where(h >= 0, h, h * pw2_ref[...])


def _fin_tc(y2, qq, dinv, b2, pw2):
    return pl.pallas_call(
        _fin_body,
        grid=(_NBLK,),
        in_specs=[
            pl.BlockSpec((_BLK, D), lambda i: (i, 0)),
            pl.BlockSpec((NC, _BLK, D), lambda i: (0, i, 0)),
            pl.BlockSpec((_BLK, 1), lambda i: (i, 0)),
            pl.BlockSpec((1, D), lambda i: (0, 0)),
            pl.BlockSpec((1, D), lambda i: (0, 0)),
        ],
        out_specs=pl.BlockSpec((_BLK, D), lambda i: (i, 0)),
        out_shape=jax.ShapeDtypeStruct((N, D), jnp.float32),
    )(y2, qq, dinv, b2, pw2)


@jax.jit
def kernel(x, edge_index, W1, b1, p1, W2, b2, p2):
    src3 = edge_index[0].astype(jnp.int32).reshape(NW, NCHUNK, K)
    dst3 = edge_index[1].astype(jnp.int32).reshape(NW, NCHUNK, K)
    zeros1 = jnp.zeros((RPT,), jnp.float32)
    zeros2 = jnp.zeros((RPT, D), jnp.float32)

    degp = _deg_sc()(dst3, zeros1)
    degs = (degp[0] + degp[1] + 1.0)[:, None]
    dinv, y1 = _prep_tc(degs, x)

    pp = _agg_sc()(y1, src3, dst3, zeros2)
    y2 = _mid_tc(y1, pp, dinv, W1, b1[None, :], p1[None, :], W2)

    qq = _agg_sc()(y2, src3, dst3, zeros2)
    return _fin_tc(y2, qq, dinv, b2[None, :], p2[None, :])
